# baseline (speedup 1.0000x reference)
import functools

import jax
import jax.numpy as jnp
from jax import lax
from jax.experimental import pallas as pl
from jax.experimental.pallas import tpu as pltpu
from jax.experimental.pallas import tpu_sc as plsc

B = 16
N = 256
C = 256
L = 16
GROUPS = C // L
GB = 8
RU = 4


@functools.cache
def _build():
  mesh = plsc.VectorSubcoreMesh(core_axis_name="c", subcore_axis_name="s",
                                num_cores=1)

  @functools.partial(
      pl.kernel,
      out_type=jax.ShapeDtypeStruct((B, C), jnp.int32),
      mesh=mesh,
      scratch_types=[
          pltpu.VMEM((N, C), jnp.float32),
          pltpu.VMEM((C,), jnp.int32),
      ],
  )
  def _argmax_sc(x_hbm, out_hbm, xbuf, obuf):
    b = lax.axis_index("s")

    pltpu.sync_copy(x_hbm.at[b], xbuf)

    for blk in range(GROUPS // GB):
      sls = [pl.ds((blk * GB + g) * L, L) for g in range(GB)]

      ninf = jnp.full((L,), -jnp.inf, jnp.float32)
      zero = jnp.zeros((L,), jnp.int32)

      @plsc.parallel_loop(0, N, 1, unroll=RU,
                          carry=((ninf,) * GB, (zero,) * GB))
      def scan(r, carry, sls=sls):
        bvs, bis = carry
        ri = jnp.zeros((L,), jnp.int32) + r
        nvs, nis = [], []
        for g in range(GB):
          v = xbuf[r, sls[g]]
          m = v > bvs[g]
          nvs.append(jnp.maximum(v, bvs[g]))
          nis.append(jnp.where(m, ri, bis[g]))
        return tuple(nvs), tuple(nis)

      bvs, bis = scan
      for g in range(GB):
        obuf[sls[g]] = bis[g]

    pltpu.sync_copy(obuf, out_hbm.at[b])

  return _argmax_sc


def kernel(x):
    idx = _build()(x)
    return idx.astype(jnp.int64)


# speedup vs baseline: 1.0516x; 1.0516x over previous
import functools

import jax
import jax.numpy as jnp
from jax import lax
from jax.experimental import pallas as pl
from jax.experimental.pallas import tpu as pltpu
from jax.experimental.pallas import tpu_sc as plsc

B = 16
N = 256
C = 256
L = 16
CW = C // 2
GB = 8
RU = 4


@functools.cache
def _build():
  mesh = plsc.VectorSubcoreMesh(core_axis_name="c", subcore_axis_name="s")

  @functools.partial(
      pl.kernel,
      out_type=jax.ShapeDtypeStruct((B, C), jnp.int32),
      mesh=mesh,
      scratch_types=[
          pltpu.VMEM((N, CW), jnp.float32),
          pltpu.VMEM((CW,), jnp.int32),
      ],
  )
  def _argmax_sc(x_hbm, out_hbm, xbuf, obuf):
    h = lax.axis_index("c")
    b = lax.axis_index("s")

    pltpu.sync_copy(x_hbm.at[b, :, pl.ds(h * CW, CW)], xbuf)

    sls = [pl.ds(g * L, L) for g in range(GB)]

    ninf = jnp.full((L,), -jnp.inf, jnp.float32)
    zero = jnp.zeros((L,), jnp.int32)

    @plsc.parallel_loop(0, N, 1, unroll=RU,
                        carry=((ninf,) * GB, (zero,) * GB))
    def scan(r, carry):
      bvs, bis = carry
      ri = jnp.zeros((L,), jnp.int32) + r
      nvs, nis = [], []
      for g in range(GB):
        v = xbuf[r, sls[g]]
        m = v > bvs[g]
        nvs.append(jnp.maximum(v, bvs[g]))
        nis.append(jnp.where(m, ri, bis[g]))
      return tuple(nvs), tuple(nis)

    bvs, bis = scan
    for g in range(GB):
      obuf[sls[g]] = bis[g]

    pltpu.sync_copy(obuf, out_hbm.at[b, pl.ds(h * CW, CW)])

  return _argmax_sc


def kernel(x):
    idx = _build()(x)
    return idx.astype(jnp.int64)
